# Initial kernel scaffold; baseline (speedup 1.0000x reference)
import jax
import jax.numpy as jnp
from jax.experimental import pallas as pl


def kernel(hidden_states, Wr, Wg, Wu, Wd):
    raise NotImplementedError("write your pallas kernel here")


# speedup vs baseline: 7.5568x; 7.5568x over previous
import functools

import jax
import jax.numpy as jnp
from jax.experimental import pallas as pl
from jax.experimental.pallas import tpu as pltpu

CHUNK = 128


def _cumsum_shift(x, axis, n):
    s = 1
    while s < n:
        if axis == 0:
            shifted = jnp.concatenate(
                [jnp.zeros((s, x.shape[1]), x.dtype), x[:-s, :]], axis=0)
        else:
            shifted = jnp.concatenate(
                [jnp.zeros((x.shape[0], s), x.dtype), x[:, :-s]], axis=1)
        x = x + shifted
        s *= 2
    return x


def _router_body(x_ref, wr_ref, pos_ref, w_ref, cnt_ref, off_ref, *, E, EP):
    x = x_ref[...]
    T = x.shape[0]
    logits = jnp.dot(x, wr_ref[...], preferred_element_type=jnp.float32)
    lane = jax.lax.broadcasted_iota(jnp.int32, logits.shape, 1)
    logits = jnp.where(lane < E, logits, -1e30)
    m = jnp.max(logits, axis=-1, keepdims=True)
    p = jnp.exp(logits - m)
    p = p / jnp.sum(p, axis=-1, keepdims=True)
    pmax = jnp.max(p, axis=-1, keepdims=True)
    e_idx = jnp.min(jnp.where(p == pmax, lane, EP), axis=-1, keepdims=True)
    onehot = (lane == e_idx).astype(jnp.float32)
    counts = jnp.sum(onehot, axis=0, keepdims=True)
    cpad = jnp.floor((counts + 7.0) / 8.0) * 8.0
    off_excl = _cumsum_shift(cpad, 1, EP) - cpad
    rank = _cumsum_shift(onehot, 0, T) - onehot
    pos = jnp.sum(onehot * (rank + off_excl), axis=-1, keepdims=True)
    pos_ref[...] = jnp.broadcast_to(pos, (T, EP)).astype(jnp.int32)
    w_ref[...] = jnp.broadcast_to(pmax, (T, EP))
    cnt_ref[...] = counts.astype(jnp.int32)
    off_ref[...] = off_excl.astype(jnp.int32)


def _permute_body(pos_row_ref, x_ref, xs_ref, *, BLK):
    i = pl.program_id(0)
    T = x_ref.shape[0]
    ids = i * BLK + jax.lax.broadcasted_iota(jnp.int32, (BLK, T), 0)
    G = (ids == pos_row_ref[...]).astype(jnp.float32)
    xs_ref[...] = jnp.dot(G, x_ref[...], preferred_element_type=jnp.float32)


def _ffn_body(xs_ref, wg_ref, wu_ref, wd_ref, off_ref, cnt_ref, y_ref, *, MAXCH):
    e = pl.program_id(0)
    off = off_ref[e]
    cnt = cnt_ref[e]
    wg = wg_ref[0]
    wu = wu_ref[0]
    wd = wd_ref[0]

    @pl.when(e == 0)
    def _zero():
        y_ref[...] = jnp.zeros_like(y_ref)

    def chunk(i, _):
        @pl.when(i * CHUNK < cnt)
        def _do():
            start = pl.multiple_of(off + i * CHUNK, 8)
            rows = xs_ref[pl.ds(start, CHUNK), :]
            gate = jnp.dot(rows, wg, preferred_element_type=jnp.float32)
            up = jnp.dot(rows, wu, preferred_element_type=jnp.float32)
            act = up * (gate * jax.nn.sigmoid(gate))
            y_ref[pl.ds(start, CHUNK), :] = jnp.dot(
                act, wd, preferred_element_type=jnp.float32)
        return 0

    jax.lax.fori_loop(0, MAXCH, chunk, 0)


def _combine_body(pos_ref, w_ref, ys_ref, out_ref, *, BLK):
    Tpad = ys_ref.shape[0]
    ids = jax.lax.broadcasted_iota(jnp.int32, (BLK, Tpad), 1)
    M = (ids == pos_ref[...][:, 0:1]).astype(jnp.float32)
    y = jnp.dot(M, ys_ref[...], preferred_element_type=jnp.float32)
    out_ref[...] = y * w_ref[...][:, 0:1]


def kernel(hidden_states, Wr, Wg, Wu, Wd):
    b, s, h = hidden_states.shape
    T = b * s
    E, H, F = Wg.shape
    EP = 128
    flat = hidden_states.reshape(T, h)
    wr_pad = jnp.zeros((H, EP), jnp.float32).at[:, :E].set(Wr)

    pos_b, w_b, cnt2d, off2d = pl.pallas_call(
        functools.partial(_router_body, E=E, EP=EP),
        out_shape=[
            jax.ShapeDtypeStruct((T, EP), jnp.int32),
            jax.ShapeDtypeStruct((T, EP), jnp.float32),
            jax.ShapeDtypeStruct((1, EP), jnp.int32),
            jax.ShapeDtypeStruct((1, EP), jnp.int32),
        ],
    )(flat, wr_pad)

    pos_1d = pos_b[:, 0]
    offs = off2d[0, :E]
    cnts = cnt2d[0, :E]

    tpad = T + E * 7 + CHUNK
    TPAD = ((tpad + 895) // 896) * 896
    PBLK = 896
    n_pblk = TPAD // PBLK

    xs = pl.pallas_call(
        functools.partial(_permute_body, BLK=PBLK),
        grid=(n_pblk,),
        in_specs=[
            pl.BlockSpec((1, T), lambda i: (0, 0)),
            pl.BlockSpec((T, H), lambda i: (0, 0)),
        ],
        out_specs=pl.BlockSpec((PBLK, H), lambda i: (i, 0)),
        out_shape=jax.ShapeDtypeStruct((TPAD, H), jnp.float32),
        compiler_params=pltpu.CompilerParams(
            dimension_semantics=("arbitrary",)),
    )(pos_1d.reshape(1, T), flat)

    MAXCH = (T + CHUNK - 1) // CHUNK
    ys = pl.pallas_call(
        functools.partial(_ffn_body, MAXCH=MAXCH),
        grid=(E,),
        in_specs=[
            pl.BlockSpec((TPAD, H), lambda e: (0, 0)),
            pl.BlockSpec((1, H, F), lambda e: (e, 0, 0)),
            pl.BlockSpec((1, H, F), lambda e: (e, 0, 0)),
            pl.BlockSpec((1, F, H), lambda e: (e, 0, 0)),
            pl.BlockSpec(memory_space=pltpu.SMEM),
            pl.BlockSpec(memory_space=pltpu.SMEM),
        ],
        out_specs=pl.BlockSpec((TPAD, H), lambda e: (0, 0)),
        out_shape=jax.ShapeDtypeStruct((TPAD, H), jnp.float32),
        compiler_params=pltpu.CompilerParams(
            dimension_semantics=("arbitrary",)),
    )(xs, Wg, Wu, Wd, offs, cnts)

    CBLK = 512
    out = pl.pallas_call(
        functools.partial(_combine_body, BLK=CBLK),
        grid=(T // CBLK,),
        in_specs=[
            pl.BlockSpec((CBLK, EP), lambda i: (i, 0)),
            pl.BlockSpec((CBLK, EP), lambda i: (i, 0)),
            pl.BlockSpec((TPAD, H), lambda i: (0, 0)),
        ],
        out_specs=pl.BlockSpec((CBLK, H), lambda i: (i, 0)),
        out_shape=jax.ShapeDtypeStruct((T, H), jnp.float32),
        compiler_params=pltpu.CompilerParams(
            dimension_semantics=("arbitrary",)),
    )(pos_b, w_b, ys)

    return out.reshape(b, s, h)
